# baseline (speedup 1.0000x reference)
import functools

import jax
import jax.numpy as jnp
from jax import lax
from jax.experimental import pallas as pl
from jax.experimental.pallas import tpu as pltpu
from jax.experimental.pallas import tpu_sc as plsc

D_MODEL = 768
NUM_EXPERTS = 8
TOP_K = 2
EXPERT_DIM = 128
EH = NUM_EXPERTS * EXPERT_DIM
TOKEN_BLOCK = 1024
NUM_WORKERS = 32
LANES = 16


def _gating_block(x_ref, gw_ref, gb_ref, out_ref):
    logits = jnp.dot(x_ref[...], gw_ref[...],
                     preferred_element_type=jnp.float32)
    out_ref[...] = logits + gb_ref[...]


def _make_router(t):
    tpw = t // NUM_WORKERS
    groups = tpw // LANES

    @functools.partial(
        pl.kernel,
        out_type=jax.ShapeDtypeStruct((t * NUM_EXPERTS,), jnp.float32),
        mesh=plsc.VectorSubcoreMesh(core_axis_name="c", subcore_axis_name="s"),
        compiler_params=pltpu.CompilerParams(needs_layout_passes=False),
        scratch_types=[
            pltpu.VMEM((tpw * NUM_EXPERTS,), jnp.float32),
            pltpu.VMEM((tpw * NUM_EXPERTS,), jnp.float32),
        ],
    )
    def _route(logits_hbm, coef_hbm, lg_v, cf_v):
        wid = lax.axis_index("s") * 2 + lax.axis_index("c")
        base = wid * tpw * NUM_EXPERTS
        pltpu.sync_copy(logits_hbm.at[pl.ds(base, tpw * NUM_EXPERTS)], lg_v)
        for k in range(tpw * NUM_EXPERTS // LANES):
            cf_v[pl.ds(k * LANES, LANES)] = jnp.zeros((LANES,), jnp.float32)
        lane = lax.broadcasted_iota(jnp.int32, (LANES,), 0)
        for g in range(groups):
            row0 = (g * LANES + lane) * NUM_EXPERTS
            l1 = plsc.load_gather(lg_v, [row0])
            i1 = jnp.zeros((LANES,), jnp.int32)
            l2 = jnp.full((LANES,), -jnp.inf, jnp.float32)
            i2 = jnp.zeros((LANES,), jnp.int32)
            for e in range(1, NUM_EXPERTS):
                v = plsc.load_gather(lg_v, [row0 + e])
                ev = jnp.full((LANES,), e, jnp.int32)
                gt1 = v > l1
                gt2 = v > l2
                new_l2 = jnp.where(gt1, l1, jnp.where(gt2, v, l2))
                new_i2 = jnp.where(gt1, i1, jnp.where(gt2, ev, i2))
                l1 = jnp.where(gt1, v, l1)
                i1 = jnp.where(gt1, ev, i1)
                l2 = new_l2
                i2 = new_i2
            r = jnp.exp(l2 - l1)
            c1 = 1.0 / (1.0 + r)
            c2 = r * c1
            plsc.store_scatter(cf_v, [row0 + i1], c1)
            plsc.store_scatter(cf_v, [row0 + i2], c2)
        pltpu.sync_copy(cf_v, coef_hbm.at[pl.ds(base, tpw * NUM_EXPERTS)])

    return _route


def _moe_block(x_ref, coef_ref, w1_ref, b1_ref, w2_ref, b2_ref, out_ref):
    xb = x_ref[...]
    coef = coef_ref[...]

    ei = lax.broadcasted_iota(jnp.int32, (NUM_EXPERTS, EH), 0)
    hi = lax.broadcasted_iota(jnp.int32, (NUM_EXPERTS, EH), 1)
    expand = (ei == (hi >> 7)).astype(jnp.float32)
    ce = jnp.dot(coef, expand, preferred_element_type=jnp.float32)

    h = jnp.dot(xb.astype(jnp.bfloat16), w1_ref[...],
                preferred_element_type=jnp.float32)
    h = jnp.maximum(h + b1_ref[...], 0.0)
    acc = jnp.dot((h * ce).astype(jnp.bfloat16), w2_ref[...],
                  preferred_element_type=jnp.float32)
    acc = acc + jnp.dot(coef, b2_ref[...], preferred_element_type=jnp.float32)
    out_ref[...] = acc


def kernel(x, gate_W, gate_b, W1, b1, W2, b2):
    batch, seq, d = x.shape
    x_flat = x.reshape(-1, d)
    t = x_flat.shape[0]
    w1a = W1.transpose(1, 0, 2).reshape(D_MODEL, EH).astype(jnp.bfloat16)
    b1a = b1.reshape(1, EH)
    w2a = W2.reshape(EH, D_MODEL)
    grid = (t // TOKEN_BLOCK,)

    logits = pl.pallas_call(
        _gating_block,
        grid=grid,
        in_specs=[
            pl.BlockSpec((TOKEN_BLOCK, D_MODEL), lambda i: (i, 0)),
            pl.BlockSpec((D_MODEL, NUM_EXPERTS), lambda i: (0, 0)),
            pl.BlockSpec((1, NUM_EXPERTS), lambda i: (0, 0)),
        ],
        out_specs=pl.BlockSpec((TOKEN_BLOCK, NUM_EXPERTS), lambda i: (i, 0)),
        out_shape=jax.ShapeDtypeStruct((t, NUM_EXPERTS), jnp.float32),
        compiler_params=pltpu.CompilerParams(
            dimension_semantics=("parallel",),
        ),
    )(x_flat, gate_W, gate_b.reshape(1, -1))

    coef = _make_router(t)(logits.reshape(-1)).reshape(t, NUM_EXPERTS)

    out = pl.pallas_call(
        _moe_block,
        grid=grid,
        in_specs=[
            pl.BlockSpec((TOKEN_BLOCK, D_MODEL), lambda i: (i, 0)),
            pl.BlockSpec((TOKEN_BLOCK, NUM_EXPERTS), lambda i: (i, 0)),
            pl.BlockSpec((D_MODEL, EH), lambda i: (0, 0)),
            pl.BlockSpec((1, EH), lambda i: (0, 0)),
            pl.BlockSpec((EH, D_MODEL), lambda i: (0, 0)),
            pl.BlockSpec((NUM_EXPERTS, D_MODEL), lambda i: (0, 0)),
        ],
        out_specs=pl.BlockSpec((TOKEN_BLOCK, D_MODEL), lambda i: (i, 0)),
        out_shape=jax.ShapeDtypeStruct((t, D_MODEL), jnp.float32),
        compiler_params=pltpu.CompilerParams(
            dimension_semantics=("parallel",),
        ),
    )(x_flat, coef, w1a, b1a, w2a.astype(jnp.bfloat16), b2)
    return out.reshape(batch, seq, d)


# speedup vs baseline: 1.9419x; 1.9419x over previous
import jax
import jax.numpy as jnp
from jax import lax
from jax.experimental import pallas as pl
from jax.experimental.pallas import tpu as pltpu

D_MODEL = 768
NUM_EXPERTS = 8
TOP_K = 2
EXPERT_DIM = 128
EH = NUM_EXPERTS * EXPERT_DIM
TOKEN_BLOCK = 1024


def _moe_block(x_ref, gw_ref, gb_ref, w1_ref, b1_ref, w2_ref, b2_ref, out_ref):
    xb = x_ref[...]
    tb = xb.shape[0]

    logits = jnp.dot(xb, gw_ref[...], preferred_element_type=jnp.float32)
    logits = logits + gb_ref[...]

    iota = lax.broadcasted_iota(jnp.int32, (tb, NUM_EXPERTS), 1)
    big = jnp.int32(NUM_EXPERTS + 1)
    l1 = jnp.max(logits, axis=-1, keepdims=True)
    idx1 = jnp.min(jnp.where(logits >= l1, iota, big), axis=-1, keepdims=True)
    sel1 = iota == idx1
    lm = jnp.where(sel1, -jnp.inf, logits)
    l2 = jnp.max(lm, axis=-1, keepdims=True)
    idx2 = jnp.min(jnp.where(lm >= l2, iota, big), axis=-1, keepdims=True)
    sel2 = iota == idx2
    r = jnp.exp(l2 - l1)
    c1 = 1.0 / (1.0 + r)
    coef = jnp.where(sel1, c1, 0.0) + jnp.where(sel2, r * c1, 0.0)

    ei = lax.broadcasted_iota(jnp.int32, (NUM_EXPERTS, EH), 0)
    hi = lax.broadcasted_iota(jnp.int32, (NUM_EXPERTS, EH), 1)
    expand = (ei == (hi >> 7)).astype(jnp.float32)
    ce = jnp.dot(coef, expand, preferred_element_type=jnp.float32)

    h = jnp.dot(xb.astype(jnp.bfloat16), w1_ref[...],
                preferred_element_type=jnp.float32)
    h = jnp.maximum(h + b1_ref[...], 0.0)
    acc = jnp.dot((h * ce).astype(jnp.bfloat16),
                  w2_ref[...].astype(jnp.bfloat16),
                  preferred_element_type=jnp.float32)
    acc = acc + jnp.dot(coef, b2_ref[...], preferred_element_type=jnp.float32)
    out_ref[...] = acc


def kernel(x, gate_W, gate_b, W1, b1, W2, b2):
    batch, seq, d = x.shape
    x_flat = x.reshape(-1, d)
    t = x_flat.shape[0]
    w1a = W1.transpose(1, 0, 2).reshape(D_MODEL, EH).astype(jnp.bfloat16)
    b1a = b1.reshape(1, EH)
    w2a = W2.reshape(EH, D_MODEL)
    grid = (t // TOKEN_BLOCK,)
    out = pl.pallas_call(
        _moe_block,
        grid=grid,
        in_specs=[
            pl.BlockSpec((TOKEN_BLOCK, D_MODEL), lambda i: (i, 0)),
            pl.BlockSpec((D_MODEL, NUM_EXPERTS), lambda i: (0, 0)),
            pl.BlockSpec((1, NUM_EXPERTS), lambda i: (0, 0)),
            pl.BlockSpec((D_MODEL, EH), lambda i: (0, 0)),
            pl.BlockSpec((1, EH), lambda i: (0, 0)),
            pl.BlockSpec((EH, D_MODEL), lambda i: (0, 0)),
            pl.BlockSpec((NUM_EXPERTS, D_MODEL), lambda i: (0, 0)),
        ],
        out_specs=pl.BlockSpec((TOKEN_BLOCK, D_MODEL), lambda i: (i, 0)),
        out_shape=jax.ShapeDtypeStruct((t, D_MODEL), jnp.float32),
        compiler_params=pltpu.CompilerParams(
            dimension_semantics=("parallel",),
        ),
    )(x_flat, gate_W, gate_b.reshape(1, -1), w1a, b1a, w2a, b2)
    return out.reshape(batch, seq, d)


# speedup vs baseline: 2.2409x; 1.1540x over previous
import jax
import jax.numpy as jnp
from jax import lax
from jax.experimental import pallas as pl
from jax.experimental.pallas import tpu as pltpu

D_MODEL = 768
NUM_EXPERTS = 8
TOP_K = 2
EXPERT_DIM = 128
EH = NUM_EXPERTS * EXPERT_DIM
TOKEN_BLOCK = 1024


def _moe_block(x_ref, gw_ref, gb_ref, w1_ref, b1_ref, w2_ref, b2_ref, out_ref):
    xb = x_ref[...]
    w1a = jnp.concatenate(
        [w1_ref[e].astype(jnp.bfloat16) for e in range(NUM_EXPERTS)], axis=1)
    tb = xb.shape[0]

    logits = jnp.dot(xb, gw_ref[...], preferred_element_type=jnp.float32)
    logits = logits + gb_ref[...]

    iota = lax.broadcasted_iota(jnp.int32, (tb, NUM_EXPERTS), 1)
    big = jnp.int32(NUM_EXPERTS + 1)
    l1 = jnp.max(logits, axis=-1, keepdims=True)
    idx1 = jnp.min(jnp.where(logits >= l1, iota, big), axis=-1, keepdims=True)
    sel1 = iota == idx1
    lm = jnp.where(sel1, -jnp.inf, logits)
    l2 = jnp.max(lm, axis=-1, keepdims=True)
    idx2 = jnp.min(jnp.where(lm >= l2, iota, big), axis=-1, keepdims=True)
    sel2 = iota == idx2
    r = jnp.exp(l2 - l1)
    c1 = 1.0 / (1.0 + r)
    coef = jnp.where(sel1, c1, 0.0) + jnp.where(sel2, r * c1, 0.0)

    ei = lax.broadcasted_iota(jnp.int32, (NUM_EXPERTS, EH), 0)
    hi = lax.broadcasted_iota(jnp.int32, (NUM_EXPERTS, EH), 1)
    expand = (ei == (hi >> 7)).astype(jnp.float32)
    ce = jnp.dot(coef, expand, preferred_element_type=jnp.float32)

    h = jnp.dot(xb.astype(jnp.bfloat16), w1a,
                preferred_element_type=jnp.float32)
    h = jnp.maximum(h + b1_ref[...], 0.0)
    acc = jnp.dot((h * ce).astype(jnp.bfloat16),
                  w2_ref[...].astype(jnp.bfloat16),
                  preferred_element_type=jnp.float32)
    acc = acc + jnp.dot(coef, b2_ref[...], preferred_element_type=jnp.float32)
    out_ref[...] = acc


def kernel(x, gate_W, gate_b, W1, b1, W2, b2):
    batch, seq, d = x.shape
    x_flat = x.reshape(-1, d)
    t = x_flat.shape[0]
    b1a = b1.reshape(1, EH)
    w2a = W2.reshape(EH, D_MODEL)
    grid = (t // TOKEN_BLOCK,)
    out = pl.pallas_call(
        _moe_block,
        grid=grid,
        in_specs=[
            pl.BlockSpec((TOKEN_BLOCK, D_MODEL), lambda i: (i, 0)),
            pl.BlockSpec((D_MODEL, NUM_EXPERTS), lambda i: (0, 0)),
            pl.BlockSpec((1, NUM_EXPERTS), lambda i: (0, 0)),
            pl.BlockSpec((NUM_EXPERTS, D_MODEL, EXPERT_DIM), lambda i: (0, 0, 0)),
            pl.BlockSpec((1, EH), lambda i: (0, 0)),
            pl.BlockSpec((EH, D_MODEL), lambda i: (0, 0)),
            pl.BlockSpec((NUM_EXPERTS, D_MODEL), lambda i: (0, 0)),
        ],
        out_specs=pl.BlockSpec((TOKEN_BLOCK, D_MODEL), lambda i: (i, 0)),
        out_shape=jax.ShapeDtypeStruct((t, D_MODEL), jnp.float32),
        compiler_params=pltpu.CompilerParams(
            dimension_semantics=("parallel",),
        ),
    )(x_flat, gate_W, gate_b.reshape(1, -1), W1, b1a, w2a, b2)
    return out.reshape(batch, seq, d)
